# Initial kernel scaffold; baseline (speedup 1.0000x reference)
import jax
import jax.numpy as jnp
from jax.experimental import pallas as pl


def kernel(x, t, mask, layerid, table, W):
    raise NotImplementedError("write your pallas kernel here")


# speedup vs baseline: 91.6039x; 91.6039x over previous
import functools

import numpy as np
import jax
import jax.numpy as jnp
from jax import lax
from jax.experimental import pallas as pl
from jax.experimental.pallas import tpu as pltpu
from jax.experimental.pallas import tpu_sc as plsc

_GRID_LEVELS = 16
_BANK_DIM = 2
_TBL = 2 ** 16
_COARSE = 16
_FINE = 512
_FEAT_DIM = 16
_BBOX_MIN = np.array([-1.0, 0.0, 0.0], dtype=np.float32)
_BBOX_MAX = np.array([1.0, 2.0, 1.0], dtype=np.float32)
_B_GROWTH = float(np.exp(np.log(_FINE / _COARSE) / (_GRID_LEVELS - 1)))
_RES = [int(np.floor(_COARSE * (_B_GROWTH ** l))) for l in range(_GRID_LEVELS)]
_P1_I32 = int(np.int64(2654435761) - (1 << 32))
_P2_I32 = 805459861

_NC = 2
_NS = 16
_NW = _NC * _NS
_CHUNK = 4096


def _sc_body(xn_hbm, tbl_hbm, res_hbm, out_hbm, u_v, v_v, w_v, o_v, t_v, r_v):
    wid = lax.axis_index("s") * _NC + lax.axis_index("c")
    lvl = wid % _GRID_LEVELS
    half = wid // _GRID_LEVELS
    p_total = xn_hbm.shape[0] // 3
    ppw = p_total // 2
    base = half * ppw

    pltpu.sync_copy(tbl_hbm.at[pl.ds(lvl * _TBL, _TBL)], t_v)
    pltpu.sync_copy(res_hbm.at[pl.ds(lvl * 16, 16)], r_v)
    res_f = r_v[...]

    nchunks = ppw // _CHUNK

    def chunk_body(g, _):
        off = base + g * _CHUNK
        pltpu.sync_copy(xn_hbm.at[pl.ds(off, _CHUNK)], u_v)
        pltpu.sync_copy(xn_hbm.at[pl.ds(p_total + off, _CHUNK)], v_v)
        pltpu.sync_copy(xn_hbm.at[pl.ds(2 * p_total + off, _CHUNK)], w_v)

        def vec_body(i, _):
            b = i * 16
            pu = u_v[pl.ds(b, 16)] * res_f
            pv = v_v[pl.ds(b, 16)] * res_f
            pw = w_v[pl.ds(b, 16)] * res_f
            iu = pu.astype(jnp.int32)
            iv = pv.astype(jnp.int32)
            iw = pw.astype(jnp.int32)
            fu = pu - iu.astype(jnp.float32)
            fv = pv - iv.astype(jnp.float32)
            fw = pw - iw.astype(jnp.float32)
            m0 = iu & (_TBL - 1)
            m0p = (iu + 1) & (_TBL - 1)
            h1 = iv * _P1_I32
            m1 = h1 & (_TBL - 1)
            m1p = (h1 + _P1_I32) & (_TBL - 1)
            h2 = iw * _P2_I32
            m2 = h2 & (_TBL - 1)
            m2p = (h2 + _P2_I32) & (_TBL - 1)
            x00 = m0 ^ m1
            x10 = m0p ^ m1
            x01 = m0 ^ m1p
            x11 = m0p ^ m1p
            wu0 = 1.0 - fu
            wv0 = 1.0 - fv
            ww0 = 1.0 - fw
            a00 = wu0 * wv0
            a10 = fu * wv0
            a01 = wu0 * fv
            a11 = fu * fv
            acc0 = jnp.zeros((16,), jnp.float32)
            acc1 = jnp.zeros((16,), jnp.float32)
            for (xy, axy) in ((x00, a00), (x10, a10), (x01, a01), (x11, a11)):
                for (mz, wz) in ((m2, ww0), (m2p, fw)):
                    g16 = plsc.load_gather(t_v, [xy ^ mz])
                    b0 = plsc.bitcast(g16 << 16, jnp.float32)
                    b1 = plsc.bitcast(g16 & (-65536), jnp.float32)
                    wc = axy * wz
                    acc0 = acc0 + wc * b0
                    acc1 = acc1 + wc * b1
            word = plsc.bitcast(
                plsc.pack(acc0, acc1, format=plsc.PackFormat.INTERLEAVED),
                jnp.int32)
            o_v[pl.ds(b, 16)] = word
            return 0

        lax.fori_loop(0, _CHUNK // 16, vec_body, 0)
        pltpu.sync_copy(o_v, out_hbm.at[pl.ds(lvl * p_total + off, _CHUNK)])
        return 0

    lax.fori_loop(0, nchunks, chunk_body, 0)


def _sc_interp(xn_flat, packed_tbl_flat, res_flat):
    p_total = xn_flat.shape[0] // 3
    mesh = plsc.VectorSubcoreMesh(core_axis_name="c", subcore_axis_name="s")
    return pl.kernel(
        _sc_body,
        out_type=jax.ShapeDtypeStruct((_GRID_LEVELS * p_total,), jnp.int32),
        mesh=mesh,
        compiler_params=pltpu.CompilerParams(needs_layout_passes=False),
        scratch_types=[
            pltpu.VMEM((_CHUNK,), jnp.float32),
            pltpu.VMEM((_CHUNK,), jnp.float32),
            pltpu.VMEM((_CHUNK,), jnp.float32),
            pltpu.VMEM((_CHUNK,), jnp.int32),
            pltpu.VMEM((_TBL,), jnp.int32),
            pltpu.VMEM((16,), jnp.float32),
        ],
    )(xn_flat, packed_tbl_flat, res_flat)


_TC_CHUNK = 2048


def _tc_body(pk_ref, pts_ref, w0_ref, w1_ref, scale_ref, off_ref, out_ref):
    pk = pk_ref[...]
    f0 = lax.bitcast_convert_type(pk << 16, jnp.float32)
    f1 = lax.bitcast_convert_type(pk & (-65536), jnp.float32)
    feat = lax.dot_general(
        f0, w0_ref[...], (((0,), (0,)), ((), ())),
        preferred_element_type=jnp.float32)
    feat = feat + lax.dot_general(
        f1, w1_ref[...], (((0,), (0,)), ((), ())),
        preferred_element_type=jnp.float32)
    p = pts_ref[...]
    z = jnp.concatenate([p] * 12, axis=1) * scale_ref[...] + off_ref[...]
    trig = jnp.sin(z)
    out_ref[...] = jnp.concatenate([feat, p, trig], axis=1)


def _tc_combine(packed_acc, pts, w0, w1, scale, off):
    p_total = pts.shape[0]
    grid = (p_total // _TC_CHUNK,)
    return pl.pallas_call(
        _tc_body,
        grid=grid,
        in_specs=[
            pl.BlockSpec((_GRID_LEVELS, _TC_CHUNK), lambda i: (0, i)),
            pl.BlockSpec((_TC_CHUNK, 3), lambda i: (i, 0)),
            pl.BlockSpec((_GRID_LEVELS, _FEAT_DIM), lambda i: (0, 0)),
            pl.BlockSpec((_GRID_LEVELS, _FEAT_DIM), lambda i: (0, 0)),
            pl.BlockSpec((1, 36), lambda i: (0, 0)),
            pl.BlockSpec((1, 36), lambda i: (0, 0)),
        ],
        out_specs=pl.BlockSpec((_TC_CHUNK, 55), lambda i: (i, 0)),
        out_shape=jax.ShapeDtypeStruct((p_total, 55), jnp.float32),
    )(packed_acc, pts, w0, w1, scale, off)


def _posenc_consts():
    scale = np.zeros((1, 36), np.float32)
    off = np.zeros((1, 36), np.float32)
    for i in range(6):
        for k in range(6):
            j = 6 * i + k
            scale[0, j] = float((2.0 ** i) * np.pi)
            off[0, j] = 0.0 if k < 3 else float(np.pi / 2)
    return scale, off


_SCALE_NP, _OFF_NP = _posenc_consts()


def kernel(x, t, mask, layerid, table, W):
    n, s, _ = x.shape
    p_total = n * s
    sel = mask[1:].astype(jnp.int32)
    xs = jnp.take(x, sel, axis=-1)
    tt = jnp.broadcast_to(t[:, None, :], (n, s, 1))
    pts = jnp.concatenate([xs, tt], axis=-1).reshape(p_total, 3)
    xn = (pts - _BBOX_MIN) / (_BBOX_MAX - _BBOX_MIN)
    xn_flat = xn.T.reshape(-1)

    tb16 = lax.bitcast_convert_type(
        table.astype(jnp.bfloat16), jnp.uint16).astype(jnp.uint32)
    packed_tbl = lax.bitcast_convert_type(
        tb16[..., 0] | (tb16[..., 1] << 16), jnp.int32).reshape(-1)

    res_b = jnp.asarray(
        np.broadcast_to(
            np.array(_RES, np.float32)[:, None], (_GRID_LEVELS, 16)).reshape(-1).copy())

    acc = _sc_interp(xn_flat, packed_tbl, res_b)
    acc = acc.reshape(_GRID_LEVELS, p_total)

    w0 = W[0::2]
    w1 = W[1::2]
    latent = _tc_combine(acc, pts, w0, w1,
                         jnp.asarray(_SCALE_NP), jnp.asarray(_OFF_NP))
    return latent.reshape(n, s, 55)
